# Initial kernel scaffold; baseline (speedup 1.0000x reference)
import jax
import jax.numpy as jnp
from jax.experimental import pallas as pl


def kernel(edge_index, paper_edge_index, author_edge_index, x_s, x_t, Wls, bls, Wlt, blt, W1, a1s, a1d, W2, a2s, a2d):
    raise NotImplementedError("write your pallas kernel here")


# speedup vs baseline: 1.0350x; 1.0350x over previous
import functools

import jax
import jax.numpy as jnp
from jax import lax
from jax.experimental import pallas as pl
from jax.experimental.pallas import tpu as pltpu

_D = 256
_BLK = 1000


def _dense_body(x_ref, wl_ref, bl_ref, w1_ref, w2_ref, at_ref,
                h1_ref, h2_ref, a_ref):
    x = x_ref[...]
    wl = wl_ref[0]
    bl = bl_ref[0]
    t = jnp.dot(x, wl, preferred_element_type=jnp.float32) + bl
    h1 = jnp.dot(t, w1_ref[...], preferred_element_type=jnp.float32)
    h2 = jnp.dot(t, w2_ref[...], preferred_element_type=jnp.float32)
    h1_ref[...] = h1
    h2_ref[...] = h2
    at = at_ref[...]
    a12 = jnp.dot(h1, at[:, :2], preferred_element_type=jnp.float32)
    a34 = jnp.dot(h2, at[:, 2:], preferred_element_type=jnp.float32)
    a_ref[...] = jnp.concatenate([a12, a34], axis=1)


def _dense_stage(xcat, wl_st, bl_st, w1, w2, at, n_s):
    n = xcat.shape[0]
    nblocks = n // _BLK
    nbs = n_s // _BLK
    grid = (nblocks,)
    return pl.pallas_call(
        _dense_body,
        grid=grid,
        in_specs=[
            pl.BlockSpec((_BLK, _D), lambda i: (i, 0)),
            pl.BlockSpec((1, _D, _D), lambda i: (i // nbs, 0, 0)),
            pl.BlockSpec((1, 1, _D), lambda i: (i // nbs, 0, 0)),
            pl.BlockSpec((_D, _D), lambda i: (0, 0)),
            pl.BlockSpec((_D, _D), lambda i: (0, 0)),
            pl.BlockSpec((_D, 4), lambda i: (0, 0)),
        ],
        out_specs=[
            pl.BlockSpec((_BLK, _D), lambda i: (i, 0)),
            pl.BlockSpec((_BLK, _D), lambda i: (i, 0)),
            pl.BlockSpec((_BLK, 4), lambda i: (i, 0)),
        ],
        out_shape=[
            jax.ShapeDtypeStruct((n, _D), jnp.float32),
            jax.ShapeDtypeStruct((n, _D), jnp.float32),
            jax.ShapeDtypeStruct((n, 4), jnp.float32),
        ],
    )(xcat, wl_st, bl_st, w1, w2, at)


def _edge_phase(h, a_s, a_d, src, dst, num_nodes):
    e = jax.nn.leaky_relu(a_s[src] + a_d[dst], 0.2)
    m = jax.ops.segment_max(e, dst, num_segments=num_nodes)
    m = jnp.where(jnp.isfinite(m), m, 0.0)
    ex = jnp.exp(e - m[dst])
    den = jax.ops.segment_sum(ex, dst, num_segments=num_nodes)
    alpha = ex / (den[dst] + 1e-16)
    return jax.ops.segment_sum(h[src] * alpha[:, None], dst,
                               num_segments=num_nodes)


def kernel(edge_index, paper_edge_index, author_edge_index, x_s, x_t,
           Wls, bls, Wlt, blt, W1, a1s, a1d, W2, a2s, a2d):
    n_s = x_s.shape[0]
    n = n_s + x_t.shape[0]

    xcat = jnp.concatenate([x_s, x_t], axis=0)
    wl_st = jnp.stack([Wls, Wlt])
    bl_st = jnp.stack([bls[None, :], blt[None, :]])
    at = jnp.stack([a1s, a1d, a2s, a2d], axis=1)

    h1, h2, a = _dense_stage(xcat, wl_st, bl_st, W1, W2, at, n_s)

    src = edge_index[0]
    dst = edge_index[1] + n_s
    loops = jnp.arange(n, dtype=src.dtype)

    e1_src = jnp.concatenate([src, author_edge_index[0], loops])
    e1_dst = jnp.concatenate([dst, author_edge_index[1], loops])
    out1 = jax.nn.relu(_edge_phase(h1, a[:, 0], a[:, 1], e1_src, e1_dst, n))

    e2_src = jnp.concatenate([dst, paper_edge_index[0], loops])
    e2_dst = jnp.concatenate([src, paper_edge_index[1], loops])
    out2 = jax.nn.relu(_edge_phase(h2, a[:, 2], a[:, 3], e2_src, e2_dst, n))

    return (out2[:n_s], out1[n_s:])


# speedup vs baseline: 2.2033x; 2.1288x over previous
import functools

import jax
import jax.numpy as jnp
from jax import lax
from jax.experimental import pallas as pl
from jax.experimental.pallas import tpu as pltpu
from jax.experimental.pallas import tpu_sc as plsc

_D = 256
_BLK = 1000
_N = 20000
_NPAD = 20480
_EP = 340992
_EPT = _EP // 16
_CH = 5120
_CSZ = 576
_B = 96
_NG = _CSZ // _B
_NCH = _EPT // _CSZ
_RPT = _NPAD // 16


def _dense_body(x_ref, wl_ref, bl_ref, w1_ref, w2_ref, at_ref,
                h1_ref, h2_ref, a_ref):
    x = x_ref[...]
    t = jnp.dot(x, wl_ref[0], preferred_element_type=jnp.float32) + bl_ref[0]
    h1 = jnp.dot(t, w1_ref[...], preferred_element_type=jnp.float32)
    h2 = jnp.dot(t, w2_ref[...], preferred_element_type=jnp.float32)
    h1_ref[...] = h1
    h2_ref[...] = h2
    at = at_ref[...]
    a12 = jnp.dot(h1, at[:, :2], preferred_element_type=jnp.float32)
    a34 = jnp.dot(h2, at[:, 2:], preferred_element_type=jnp.float32)
    a_ref[...] = jnp.concatenate([a12, a34], axis=1)


def _dense_stage(xcat, wl_st, bl_st, w1, w2, at, n_s):
    n = xcat.shape[0]
    nbs = n_s // _BLK
    return pl.pallas_call(
        _dense_body,
        grid=(n // _BLK,),
        in_specs=[
            pl.BlockSpec((_BLK, _D), lambda i: (i, 0)),
            pl.BlockSpec((1, _D, _D), lambda i: (i // nbs, 0, 0)),
            pl.BlockSpec((1, 1, _D), lambda i: (i // nbs, 0, 0)),
            pl.BlockSpec((_D, _D), lambda i: (0, 0)),
            pl.BlockSpec((_D, _D), lambda i: (0, 0)),
            pl.BlockSpec((_D, 4), lambda i: (0, 0)),
        ],
        out_specs=[
            pl.BlockSpec((_BLK, _D), lambda i: (i, 0)),
            pl.BlockSpec((_BLK, _D), lambda i: (i, 0)),
            pl.BlockSpec((_BLK, 4), lambda i: (i, 0)),
        ],
        out_shape=[
            jax.ShapeDtypeStruct((n, _D), jnp.float32),
            jax.ShapeDtypeStruct((n, _D), jnp.float32),
            jax.ShapeDtypeStruct((n, 4), jnp.float32),
        ],
    )(xcat, wl_st, bl_st, w1, w2, at)


def _sc_conv(sd_hbm, asv_hbm, adv_hbm, hlo_hbm, hhi_hbm, zrows_hbm, zflat_hbm,
             out_hbm, den_hbm, ex_hbm,
             sdbuf, exbuf, asv_sh, adv_sh, den_sh, acc,
             sidx, didx, gidx, exb, asg, adg, sjdx, rbt, sem):
    c = lax.axis_index("c")
    s = lax.axis_index("s")

    sl = pl.ds(s * _RPT, _RPT)
    pltpu.sync_copy(asv_hbm.at[sl], asv_sh.at[sl])
    pltpu.sync_copy(adv_hbm.at[sl], adv_sh.at[sl])
    pltpu.sync_copy(zflat_hbm, den_sh.at[sl])
    plsc.subcore_barrier()

    def _chunk1(ci, carry):
        eoff = s * _EPT + ci * _CSZ
        pltpu.sync_copy(sd_hbm.at[pl.ds(eoff, _CSZ)], sdbuf)
        for v in range(_NG):
            goff = v * _B
            for w in range(_B // 16):
                sd16 = sdbuf[pl.ds(goff + w * 16, 16)]
                sidx[pl.ds(w * 16, 16)] = jnp.bitwise_and(sd16, 0xFFFF)
                didx[pl.ds(w * 16, 16)] = lax.shift_right_logical(sd16, 16)
            pltpu.async_copy(asv_sh.at[sidx], asg, sem).wait()
            pltpu.async_copy(adv_sh.at[didx], adg, sem).wait()
            for w in range(_B // 16):
                e = asg[pl.ds(w * 16, 16)] + adg[pl.ds(w * 16, 16)]
                e = jnp.where(e < 0.0, e * 0.2, e)
                exbuf[pl.ds(goff + w * 16, 16)] = jnp.exp(e)

            @pl.when(c == (v & 1))
            def _den():
                pltpu.sync_copy(exbuf.at[pl.ds(goff, _B)],
                                den_sh.at[didx], add=True)
        pltpu.sync_copy(exbuf, ex_hbm.at[pl.ds(eoff, _CSZ)])
        return carry
    lax.fori_loop(0, _NCH, _chunk1, 0)

    plsc.subcore_barrier()

    pltpu.sync_copy(den_sh.at[sl],
                    den_hbm.at[pl.ds(c * _NPAD + s * _RPT, _RPT)])

    def _pass(base):
        pltpu.sync_copy(zrows_hbm, acc.at[pl.ds(s * (_CH // 8), _CH // 8)])
        plsc.subcore_barrier()

        def _chunk2(ci, carry):
            eoff = s * _EPT + ci * _CSZ
            pltpu.sync_copy(sd_hbm.at[pl.ds(eoff, _CSZ)], sdbuf)
            pltpu.sync_copy(ex_hbm.at[pl.ds(eoff, _CSZ)], exbuf)
            for v in range(_NG):
                goff = v * _B
                for w in range(_B // 16):
                    sd16 = sdbuf[pl.ds(goff + w * 16, 16)]
                    s16 = jnp.bitwise_and(sd16, 0xFFFF)
                    d16 = lax.shift_right_logical(sd16, 16)
                    lo = d16 - base
                    inr = jnp.logical_and(lo >= 0, lo < _CH)
                    gidx[pl.ds(w * 16, 16)] = s16
                    sidx[pl.ds(w * 16, 16)] = jnp.where(inr, lo, 0)
                    ex16 = exbuf[pl.ds(goff + w * 16, 16)]
                    exb[pl.ds(w * 16, 16)] = jnp.where(inr, ex16, 0.0)
                cp_lo = pltpu.async_copy(hlo_hbm.at[gidx],
                                         rbt.at[pl.ds(0, _B)], sem)
                cp_hi = pltpu.async_copy(hhi_hbm.at[gidx],
                                         rbt.at[pl.ds(_B, _B)], sem)
                cp_lo.wait()
                cp_hi.wait()

                ci16 = lax.iota(jnp.int32, 16)

                def _scale(r, cc):
                    ridx = jnp.full((16,), r, jnp.int32)
                    ex_s = plsc.load_gather(exb, [ridx])
                    for j in range(16):
                        rj = ridx + ((j // 8) * _B)
                        cidx = ci16 + ((j % 8) * 16)
                        v2 = plsc.load_gather(rbt, [rj, cidx])
                        plsc.store_scatter(rbt, [rj, cidx], v2 * ex_s)
                    return cc
                lax.fori_loop(0, _B, _scale, 0)

                for j in range(2):
                    for w in range(_B // 16):
                        lo16 = sidx[pl.ds(w * 16, 16)]
                        sjdx[pl.ds(w * 16, 16)] = lo16 * 2 + j
                    pltpu.sync_copy(rbt.at[pl.ds(j * _B, _B)],
                                    acc.at[sjdx], add=True)
            return carry
        lax.fori_loop(0, _NCH, _chunk2, 0)
        plsc.subcore_barrier()

        d0 = s * (_CH // 16)
        for o in range(0, _CH // 16, 64):
            pltpu.sync_copy(acc.at[pl.ds((d0 + o) * 2, 64 * 2)],
                            out_hbm.at[pl.ds((base + d0 + o) * 2, 64 * 2)])
        plsc.subcore_barrier()

    _pass((2 * c + 0) * _CH)
    _pass((2 * c + 1) * _CH)


@functools.partial(
    pl.kernel,
    mesh=plsc.VectorSubcoreMesh(core_axis_name="c", subcore_axis_name="s"),
    compiler_params=pltpu.CompilerParams(needs_layout_passes=False),
    out_type=[
        jax.ShapeDtypeStruct((_NPAD * 2, 128), jnp.float32),
        jax.ShapeDtypeStruct((2 * _NPAD,), jnp.float32),
        jax.ShapeDtypeStruct((_EP,), jnp.float32),
        jax.ShapeDtypeStruct((_NPAD * 2, 128), jnp.float32),
        jax.ShapeDtypeStruct((2 * _NPAD,), jnp.float32),
        jax.ShapeDtypeStruct((_EP,), jnp.float32),
    ],
    scratch_types=[
        pltpu.VMEM((_CSZ,), jnp.int32),
        pltpu.VMEM((_CSZ,), jnp.float32),
        pltpu.VMEM_SHARED((_NPAD,), jnp.float32),
        pltpu.VMEM_SHARED((_NPAD,), jnp.float32),
        pltpu.VMEM_SHARED((_NPAD,), jnp.float32),
        pltpu.VMEM_SHARED((_CH * 2, 128), jnp.float32),
        pltpu.VMEM((_B,), jnp.int32),
        pltpu.VMEM((_B,), jnp.int32),
        pltpu.VMEM((_B,), jnp.int32),
        pltpu.VMEM((_B,), jnp.float32),
        pltpu.VMEM((_B,), jnp.float32),
        pltpu.VMEM((_B,), jnp.float32),
        pltpu.VMEM((_B,), jnp.int32),
        pltpu.VMEM((2 * _B, 128), jnp.float32),
        pltpu.SemaphoreType.DMA,
    ],
)
def _sc_edge(sd1, asv1, adv1, hlo1, hhi1, sd2, asv2, adv2, hlo2, hhi2,
             zrows, zflat, out1, den1, ex1, out2, den2, ex2, *rest):
    _sc_conv(sd1, asv1, adv1, hlo1, hhi1, zrows, zflat, out1, den1, ex1,
             *rest)
    plsc.subcore_barrier()
    _sc_conv(sd2, asv2, adv2, hlo2, hhi2, zrows, zflat, out2, den2, ex2,
             *rest)


def _finish_body(unsc_ref, den_ref, o_ref):
    den = den_ref[...]
    rec = 1.0 / (den[:, 0] + den[:, 1] + 1e-16)
    o_ref[...] = jnp.maximum(unsc_ref[...] * rec[:, None], 0.0)


def _finish(unsc, den_t, row_off, rows):
    off_b = row_off // _BLK
    return pl.pallas_call(
        _finish_body,
        grid=(rows // _BLK,),
        in_specs=[
            pl.BlockSpec((_BLK, _D), lambda i: (i + off_b, 0)),
            pl.BlockSpec((_BLK, 2), lambda i: (i + off_b, 0)),
        ],
        out_specs=pl.BlockSpec((_BLK, _D), lambda i: (i, 0)),
        out_shape=jax.ShapeDtypeStruct((rows, _D), jnp.float32),
    )(unsc, den_t)


def kernel(edge_index, paper_edge_index, author_edge_index, x_s, x_t,
           Wls, bls, Wlt, blt, W1, a1s, a1d, W2, a2s, a2d):
    n_s = x_s.shape[0]
    n = n_s + x_t.shape[0]

    xcat = jnp.concatenate([x_s, x_t], axis=0)
    wl_st = jnp.stack([Wls, Wlt])
    bl_st = jnp.stack([bls[None, :], blt[None, :]])
    at = jnp.stack([a1s, a1d, a2s, a2d], axis=1)

    h1, h2, a = _dense_stage(xcat, wl_st, bl_st, W1, W2, at, n_s)

    src = edge_index[0]
    dst = edge_index[1] + n_s
    loops = jnp.arange(n, dtype=jnp.int32)

    def pack_edges(e_src, e_dst):
        npad = _EP - e_src.shape[0]
        pad_s = jnp.zeros((npad,), jnp.int32)
        pad_d = jnp.full((npad,), _N, jnp.int32)
        es = jnp.concatenate([e_src.astype(jnp.int32), pad_s])
        ed = jnp.concatenate([e_dst.astype(jnp.int32), pad_d])
        return jnp.bitwise_or(es, ed << 16)

    def pad_scores(v):
        return jnp.concatenate([v, jnp.zeros((_NPAD - n,), jnp.float32)])

    sd1 = pack_edges(jnp.concatenate([src, author_edge_index[0], loops]),
                     jnp.concatenate([dst, author_edge_index[1], loops]))
    sd2 = pack_edges(jnp.concatenate([dst, paper_edge_index[0], loops]),
                     jnp.concatenate([src, paper_edge_index[1], loops]))

    zrows = jnp.zeros((_CH // 8, 128), jnp.float32)
    zflat = jnp.zeros((_NPAD // 16,), jnp.float32)
    out1, den1, _, out2, den2, _ = _sc_edge(
        sd1, pad_scores(a[:, 0]), pad_scores(a[:, 1]),
        h1[:, :128], h1[:, 128:],
        sd2, pad_scores(a[:, 2]), pad_scores(a[:, 3]),
        h2[:, :128], h2[:, 128:], zrows, zflat)

    new_x_s = _finish(out2.reshape(_NPAD, _D), den2.reshape(2, _NPAD).T,
                      0, n_s)
    new_x_t = _finish(out1.reshape(_NPAD, _D), den1.reshape(2, _NPAD).T,
                      n_s, n - n_s)
    return (new_x_s, new_x_t)


# speedup vs baseline: 2.2208x; 1.0079x over previous
import functools

import jax
import jax.numpy as jnp
from jax import lax
from jax.experimental import pallas as pl
from jax.experimental.pallas import tpu as pltpu
from jax.experimental.pallas import tpu_sc as plsc

_D = 256
_BLK = 1000
_N = 20000
_NPAD = 20480
_EP = 340992
_EPT = _EP // 16
_CH = 5120
_CSZ = 576
_B = 96
_NG = _CSZ // _B
_NCH = _EPT // _CSZ
_RPT = _NPAD // 16


def _dense_body(x_ref, wl_ref, bl_ref, w1_ref, w2_ref, at_ref,
                h1_ref, h2_ref, a_ref):
    x = x_ref[...]
    t = jnp.dot(x, wl_ref[0], preferred_element_type=jnp.float32) + bl_ref[0]
    h1 = jnp.dot(t, w1_ref[...], preferred_element_type=jnp.float32)
    h2 = jnp.dot(t, w2_ref[...], preferred_element_type=jnp.float32)
    h1_ref[...] = h1
    h2_ref[...] = h2
    at = at_ref[...]
    a12 = jnp.dot(h1, at[:, :2], preferred_element_type=jnp.float32)
    a34 = jnp.dot(h2, at[:, 2:], preferred_element_type=jnp.float32)
    a_ref[...] = jnp.concatenate([a12, a34], axis=1)


def _dense_stage(xcat, wl_st, bl_st, w1, w2, at, n_s):
    n = xcat.shape[0]
    nbs = n_s // _BLK
    return pl.pallas_call(
        _dense_body,
        grid=(n // _BLK,),
        in_specs=[
            pl.BlockSpec((_BLK, _D), lambda i: (i, 0)),
            pl.BlockSpec((1, _D, _D), lambda i: (i // nbs, 0, 0)),
            pl.BlockSpec((1, 1, _D), lambda i: (i // nbs, 0, 0)),
            pl.BlockSpec((_D, _D), lambda i: (0, 0)),
            pl.BlockSpec((_D, _D), lambda i: (0, 0)),
            pl.BlockSpec((_D, 4), lambda i: (0, 0)),
        ],
        out_specs=[
            pl.BlockSpec((_BLK, _D), lambda i: (i, 0)),
            pl.BlockSpec((_BLK, _D), lambda i: (i, 0)),
            pl.BlockSpec((_BLK, 4), lambda i: (i, 0)),
        ],
        out_shape=[
            jax.ShapeDtypeStruct((n, _D), jnp.float32),
            jax.ShapeDtypeStruct((n, _D), jnp.float32),
            jax.ShapeDtypeStruct((n, 4), jnp.float32),
        ],
    )(xcat, wl_st, bl_st, w1, w2, at)


def _sc_conv(sd_hbm, asv_hbm, adv_hbm, hlo_hbm, hhi_hbm, zrows_hbm, zflat_hbm,
             out_hbm, den_hbm, ex_hbm,
             sdbuf, exbuf, asv_sh, adv_sh, den_sh, acc,
             sidx, didx, gidx, exb, asg, adg, sjdx, rbt, sem):
    c = lax.axis_index("c")
    s = lax.axis_index("s")

    sl = pl.ds(s * _RPT, _RPT)
    pltpu.sync_copy(asv_hbm.at[sl], asv_sh.at[sl])
    pltpu.sync_copy(adv_hbm.at[sl], adv_sh.at[sl])
    pltpu.sync_copy(zflat_hbm, den_sh.at[sl])
    plsc.subcore_barrier()

    def _chunk1(ci, carry):
        eoff = s * _EPT + ci * _CSZ
        pltpu.sync_copy(sd_hbm.at[pl.ds(eoff, _CSZ)], sdbuf)
        for v in range(_NG):
            goff = v * _B
            for w in range(_B // 16):
                sd16 = sdbuf[pl.ds(goff + w * 16, 16)]
                sidx[pl.ds(w * 16, 16)] = jnp.bitwise_and(sd16, 0xFFFF)
                didx[pl.ds(w * 16, 16)] = lax.shift_right_logical(sd16, 16)
            cp_a = pltpu.async_copy(asv_sh.at[sidx], asg, sem)
            cp_b = pltpu.async_copy(adv_sh.at[didx], adg, sem)
            cp_a.wait()
            cp_b.wait()
            for w in range(_B // 16):
                e = asg[pl.ds(w * 16, 16)] + adg[pl.ds(w * 16, 16)]
                e = jnp.where(e < 0.0, e * 0.2, e)
                exbuf[pl.ds(goff + w * 16, 16)] = jnp.exp(e)

            @pl.when(c == (v & 1))
            def _den():
                pltpu.sync_copy(exbuf.at[pl.ds(goff, _B)],
                                den_sh.at[didx], add=True)
        pltpu.sync_copy(exbuf, ex_hbm.at[pl.ds(eoff, _CSZ)])
        return carry
    lax.fori_loop(0, _NCH, _chunk1, 0)

    plsc.subcore_barrier()

    pltpu.sync_copy(den_sh.at[sl],
                    den_hbm.at[pl.ds(c * _NPAD + s * _RPT, _RPT)])

    def _pass(base):
        pltpu.sync_copy(zrows_hbm, acc.at[pl.ds(s * (_CH // 8), _CH // 8)])
        plsc.subcore_barrier()

        def _chunk2(ci, carry):
            eoff = s * _EPT + ci * _CSZ
            pltpu.sync_copy(sd_hbm.at[pl.ds(eoff, _CSZ)], sdbuf)
            pltpu.sync_copy(ex_hbm.at[pl.ds(eoff, _CSZ)], exbuf)
            for v in range(_NG):
                goff = v * _B
                for w in range(_B // 16):
                    sd16 = sdbuf[pl.ds(goff + w * 16, 16)]
                    s16 = jnp.bitwise_and(sd16, 0xFFFF)
                    d16 = lax.shift_right_logical(sd16, 16)
                    lo = d16 - base
                    inr = jnp.logical_and(lo >= 0, lo < _CH)
                    gidx[pl.ds(w * 16, 16)] = s16
                    sidx[pl.ds(w * 16, 16)] = jnp.where(inr, lo, 0)
                    ex16 = exbuf[pl.ds(goff + w * 16, 16)]
                    exb[pl.ds(w * 16, 16)] = jnp.where(inr, ex16, 0.0)
                cp_lo = pltpu.async_copy(hlo_hbm.at[gidx],
                                         rbt.at[pl.ds(0, _B)], sem)
                cp_hi = pltpu.async_copy(hhi_hbm.at[gidx],
                                         rbt.at[pl.ds(_B, _B)], sem)
                cp_lo.wait()
                cp_hi.wait()

                ci16 = lax.iota(jnp.int32, 16)

                def _scale(r, cc):
                    ridx = jnp.full((16,), r, jnp.int32)
                    ex_s = plsc.load_gather(exb, [ridx])
                    for j in range(16):
                        rj = ridx + ((j // 8) * _B)
                        cidx = ci16 + ((j % 8) * 16)
                        v2 = plsc.load_gather(rbt, [rj, cidx])
                        plsc.store_scatter(rbt, [rj, cidx], v2 * ex_s)
                    return cc
                lax.fori_loop(0, _B, _scale, 0)

                for w in range(_B // 16):
                    lo16 = sidx[pl.ds(w * 16, 16)]
                    sjdx[pl.ds(w * 16, 16)] = lo16 * 2
                    didx[pl.ds(w * 16, 16)] = lo16 * 2 + 1
                cp0 = pltpu.async_copy(rbt.at[pl.ds(0, _B)],
                                       acc.at[sjdx], sem, add=True)
                cp1 = pltpu.async_copy(rbt.at[pl.ds(_B, _B)],
                                       acc.at[didx], sem, add=True)
                cp0.wait()
                cp1.wait()
            return carry
        lax.fori_loop(0, _NCH, _chunk2, 0)
        plsc.subcore_barrier()

        d0 = s * (_CH // 16)
        for o in range(0, _CH // 16, 64):
            pltpu.sync_copy(acc.at[pl.ds((d0 + o) * 2, 64 * 2)],
                            out_hbm.at[pl.ds((base + d0 + o) * 2, 64 * 2)])
        plsc.subcore_barrier()

    _pass((2 * c + 0) * _CH)
    _pass((2 * c + 1) * _CH)


@functools.partial(
    pl.kernel,
    mesh=plsc.VectorSubcoreMesh(core_axis_name="c", subcore_axis_name="s"),
    compiler_params=pltpu.CompilerParams(needs_layout_passes=False),
    out_type=[
        jax.ShapeDtypeStruct((_NPAD * 2, 128), jnp.float32),
        jax.ShapeDtypeStruct((2 * _NPAD,), jnp.float32),
        jax.ShapeDtypeStruct((_EP,), jnp.float32),
        jax.ShapeDtypeStruct((_NPAD * 2, 128), jnp.float32),
        jax.ShapeDtypeStruct((2 * _NPAD,), jnp.float32),
        jax.ShapeDtypeStruct((_EP,), jnp.float32),
    ],
    scratch_types=[
        pltpu.VMEM((_CSZ,), jnp.int32),
        pltpu.VMEM((_CSZ,), jnp.float32),
        pltpu.VMEM_SHARED((_NPAD,), jnp.float32),
        pltpu.VMEM_SHARED((_NPAD,), jnp.float32),
        pltpu.VMEM_SHARED((_NPAD,), jnp.float32),
        pltpu.VMEM_SHARED((_CH * 2, 128), jnp.float32),
        pltpu.VMEM((_B,), jnp.int32),
        pltpu.VMEM((_B,), jnp.int32),
        pltpu.VMEM((_B,), jnp.int32),
        pltpu.VMEM((_B,), jnp.float32),
        pltpu.VMEM((_B,), jnp.float32),
        pltpu.VMEM((_B,), jnp.float32),
        pltpu.VMEM((_B,), jnp.int32),
        pltpu.VMEM((2 * _B, 128), jnp.float32),
        pltpu.SemaphoreType.DMA,
    ],
)
def _sc_edge(sd1, asv1, adv1, hlo1, hhi1, sd2, asv2, adv2, hlo2, hhi2,
             zrows, zflat, out1, den1, ex1, out2, den2, ex2, *rest):
    _sc_conv(sd1, asv1, adv1, hlo1, hhi1, zrows, zflat, out1, den1, ex1,
             *rest)
    plsc.subcore_barrier()
    _sc_conv(sd2, asv2, adv2, hlo2, hhi2, zrows, zflat, out2, den2, ex2,
             *rest)


def _finish_body(unsc_ref, den_ref, o_ref):
    den = den_ref[...]
    rec = 1.0 / (den[:, 0] + den[:, 1] + 1e-16)
    o_ref[...] = jnp.maximum(unsc_ref[...] * rec[:, None], 0.0)


def _finish(unsc, den_t, row_off, rows):
    off_b = row_off // _BLK
    return pl.pallas_call(
        _finish_body,
        grid=(rows // _BLK,),
        in_specs=[
            pl.BlockSpec((_BLK, _D), lambda i: (i + off_b, 0)),
            pl.BlockSpec((_BLK, 2), lambda i: (i + off_b, 0)),
        ],
        out_specs=pl.BlockSpec((_BLK, _D), lambda i: (i, 0)),
        out_shape=jax.ShapeDtypeStruct((rows, _D), jnp.float32),
    )(unsc, den_t)


def kernel(edge_index, paper_edge_index, author_edge_index, x_s, x_t,
           Wls, bls, Wlt, blt, W1, a1s, a1d, W2, a2s, a2d):
    n_s = x_s.shape[0]
    n = n_s + x_t.shape[0]

    xcat = jnp.concatenate([x_s, x_t], axis=0)
    wl_st = jnp.stack([Wls, Wlt])
    bl_st = jnp.stack([bls[None, :], blt[None, :]])
    at = jnp.stack([a1s, a1d, a2s, a2d], axis=1)

    h1, h2, a = _dense_stage(xcat, wl_st, bl_st, W1, W2, at, n_s)

    src = edge_index[0]
    dst = edge_index[1] + n_s
    loops = jnp.arange(n, dtype=jnp.int32)

    def pack_edges(e_src, e_dst):
        npad = _EP - e_src.shape[0]
        pad_s = jnp.zeros((npad,), jnp.int32)
        pad_d = jnp.full((npad,), _N, jnp.int32)
        es = jnp.concatenate([e_src.astype(jnp.int32), pad_s])
        ed = jnp.concatenate([e_dst.astype(jnp.int32), pad_d])
        return jnp.bitwise_or(es, ed << 16)

    def pad_scores(v):
        return jnp.concatenate([v, jnp.zeros((_NPAD - n,), jnp.float32)])

    sd1 = pack_edges(jnp.concatenate([src, author_edge_index[0], loops]),
                     jnp.concatenate([dst, author_edge_index[1], loops]))
    sd2 = pack_edges(jnp.concatenate([dst, paper_edge_index[0], loops]),
                     jnp.concatenate([src, paper_edge_index[1], loops]))

    zrows = jnp.zeros((_CH // 8, 128), jnp.float32)
    zflat = jnp.zeros((_NPAD // 16,), jnp.float32)
    out1, den1, _, out2, den2, _ = _sc_edge(
        sd1, pad_scores(a[:, 0]), pad_scores(a[:, 1]),
        h1[:, :128], h1[:, 128:],
        sd2, pad_scores(a[:, 2]), pad_scores(a[:, 3]),
        h2[:, :128], h2[:, 128:], zrows, zflat)

    new_x_s = _finish(out2.reshape(_NPAD, _D), den2.reshape(2, _NPAD).T,
                      0, n_s)
    new_x_t = _finish(out1.reshape(_NPAD, _D), den1.reshape(2, _NPAD).T,
                      n_s, n - n_s)
    return (new_x_s, new_x_t)


# speedup vs baseline: 2.3637x; 1.0644x over previous
import functools

import jax
import jax.numpy as jnp
from jax import lax
from jax.experimental import pallas as pl
from jax.experimental.pallas import tpu as pltpu
from jax.experimental.pallas import tpu_sc as plsc

_D = 256
_BLK = 1000
_N = 20000
_NPAD = 20480
_EP = 340992
_EPT = _EP // 16
_CH = 5120
_CSZ = 576
_B = 96
_NG = _CSZ // _B
_NCH = _EPT // _CSZ
_RPT = _NPAD // 16


def _dense_body(x_ref, wl_ref, bl_ref, w1_ref, w2_ref, at_ref,
                h1_ref, h2_ref, a_ref):
    x = x_ref[...]
    t = jnp.dot(x, wl_ref[0], preferred_element_type=jnp.float32) + bl_ref[0]
    h1 = jnp.dot(t, w1_ref[...], preferred_element_type=jnp.float32)
    h2 = jnp.dot(t, w2_ref[...], preferred_element_type=jnp.float32)
    h1_ref[...] = h1
    h2_ref[...] = h2
    at = at_ref[...]
    a12 = jnp.dot(h1, at[:, :2], preferred_element_type=jnp.float32)
    a34 = jnp.dot(h2, at[:, 2:], preferred_element_type=jnp.float32)
    a_ref[...] = jnp.concatenate([a12, a34], axis=1)


def _dense_stage(xcat, wl_st, bl_st, w1, w2, at, n_s):
    n = xcat.shape[0]
    nbs = n_s // _BLK
    return pl.pallas_call(
        _dense_body,
        grid=(n // _BLK,),
        in_specs=[
            pl.BlockSpec((_BLK, _D), lambda i: (i, 0)),
            pl.BlockSpec((1, _D, _D), lambda i: (i // nbs, 0, 0)),
            pl.BlockSpec((1, 1, _D), lambda i: (i // nbs, 0, 0)),
            pl.BlockSpec((_D, _D), lambda i: (0, 0)),
            pl.BlockSpec((_D, _D), lambda i: (0, 0)),
            pl.BlockSpec((_D, 4), lambda i: (0, 0)),
        ],
        out_specs=[
            pl.BlockSpec((_BLK, _D), lambda i: (i, 0)),
            pl.BlockSpec((_BLK, _D), lambda i: (i, 0)),
            pl.BlockSpec((_BLK, 4), lambda i: (i, 0)),
        ],
        out_shape=[
            jax.ShapeDtypeStruct((n, _D), jnp.float32),
            jax.ShapeDtypeStruct((n, _D), jnp.float32),
            jax.ShapeDtypeStruct((n, 4), jnp.float32),
        ],
    )(xcat, wl_st, bl_st, w1, w2, at)


def _sc_conv(sd_hbm, asv_hbm, adv_hbm, hlo_hbm, hhi_hbm, zrows_hbm, zflat_hbm,
             out_hbm, den_hbm, ex_hbm,
             sdbuf, exbuf, asv_sh, adv_sh, den_sh, acc,
             sidx, didx, gidx, exb, asg, adg, sjdx, rbt, sem):
    c = lax.axis_index("c")
    s = lax.axis_index("s")

    sl = pl.ds(s * _RPT, _RPT)
    pltpu.sync_copy(asv_hbm.at[sl], asv_sh.at[sl])
    pltpu.sync_copy(adv_hbm.at[sl], adv_sh.at[sl])
    pltpu.sync_copy(zflat_hbm, den_sh.at[sl])
    plsc.subcore_barrier()

    def _chunk1(ci, carry):
        eoff = s * _EPT + ci * _CSZ
        pltpu.sync_copy(sd_hbm.at[pl.ds(eoff, _CSZ)], sdbuf)
        for v in range(_NG):
            goff = v * _B
            for w in range(_B // 16):
                sd16 = sdbuf[pl.ds(goff + w * 16, 16)]
                sidx[pl.ds(w * 16, 16)] = jnp.bitwise_and(sd16, 0xFFFF)
                didx[pl.ds(w * 16, 16)] = lax.shift_right_logical(sd16, 16)
            cp_a = pltpu.async_copy(asv_sh.at[sidx], asg, sem)
            cp_b = pltpu.async_copy(adv_sh.at[didx], adg, sem)
            cp_a.wait()
            cp_b.wait()
            for w in range(_B // 16):
                e = asg[pl.ds(w * 16, 16)] + adg[pl.ds(w * 16, 16)]
                e = jnp.where(e < 0.0, e * 0.2, e)
                exbuf[pl.ds(goff + w * 16, 16)] = jnp.exp(e)

            @pl.when(c == (v & 1))
            def _den():
                pltpu.sync_copy(exbuf.at[pl.ds(goff, _B)],
                                den_sh.at[didx], add=True)
        pltpu.sync_copy(exbuf, ex_hbm.at[pl.ds(eoff, _CSZ)])
        return carry
    lax.fori_loop(0, _NCH, _chunk1, 0)

    plsc.subcore_barrier()

    pltpu.sync_copy(den_sh.at[sl],
                    den_hbm.at[pl.ds(c * _NPAD + s * _RPT, _RPT)])

    def _pass(base):
        pltpu.sync_copy(zrows_hbm, acc.at[pl.ds(s * (_CH // 8), _CH // 8)])
        plsc.subcore_barrier()

        def _chunk2(ci, carry):
            eoff = s * _EPT + ci * _CSZ
            pltpu.sync_copy(sd_hbm.at[pl.ds(eoff, _CSZ)], sdbuf)
            pltpu.sync_copy(ex_hbm.at[pl.ds(eoff, _CSZ)], exbuf)
            for v in range(_NG):
                goff = v * _B
                for w in range(_B // 16):
                    sd16 = sdbuf[pl.ds(goff + w * 16, 16)]
                    s16 = jnp.bitwise_and(sd16, 0xFFFF)
                    d16 = lax.shift_right_logical(sd16, 16)
                    lo = d16 - base
                    inr = jnp.logical_and(lo >= 0, lo < _CH)
                    gidx[pl.ds(w * 16, 16)] = s16
                    sidx[pl.ds(w * 16, 16)] = jnp.where(inr, lo, 0)
                    ex16 = exbuf[pl.ds(goff + w * 16, 16)]
                    exb[pl.ds(w * 16, 16)] = jnp.where(inr, ex16, 0.0)
                cp_lo = pltpu.async_copy(hlo_hbm.at[gidx],
                                         rbt.at[pl.ds(0, _B)], sem)
                cp_hi = pltpu.async_copy(hhi_hbm.at[gidx],
                                         rbt.at[pl.ds(_B, _B)], sem)
                cp_lo.wait()
                cp_hi.wait()

                ci16 = lax.iota(jnp.int32, 16)

                def _scale(r2, cc):
                    for rr in range(2):
                        ridx = jnp.full((16,), r2 * 2 + rr, jnp.int32)
                        ex_s = plsc.load_gather(exb, [ridx])
                        for j in range(16):
                            rj = ridx + ((j // 8) * _B)
                            cidx = ci16 + ((j % 8) * 16)
                            v2 = plsc.load_gather(rbt, [rj, cidx])
                            plsc.store_scatter(rbt, [rj, cidx], v2 * ex_s)
                    return cc
                lax.fori_loop(0, _B // 2, _scale, 0)

                for w in range(_B // 16):
                    lo16 = sidx[pl.ds(w * 16, 16)]
                    sjdx[pl.ds(w * 16, 16)] = lo16 * 2
                    didx[pl.ds(w * 16, 16)] = lo16 * 2 + 1
                cp0 = pltpu.async_copy(rbt.at[pl.ds(0, _B)],
                                       acc.at[sjdx], sem, add=True)
                cp1 = pltpu.async_copy(rbt.at[pl.ds(_B, _B)],
                                       acc.at[didx], sem, add=True)
                cp0.wait()
                cp1.wait()
            return carry
        lax.fori_loop(0, _NCH, _chunk2, 0)
        plsc.subcore_barrier()

        d0 = s * (_CH // 16)
        for o in range(0, _CH // 16, 64):
            pltpu.sync_copy(acc.at[pl.ds((d0 + o) * 2, 64 * 2)],
                            out_hbm.at[pl.ds((base + d0 + o) * 2, 64 * 2)])
        plsc.subcore_barrier()

    _pass((2 * c + 0) * _CH)
    _pass((2 * c + 1) * _CH)


@functools.partial(
    pl.kernel,
    mesh=plsc.VectorSubcoreMesh(core_axis_name="c", subcore_axis_name="s"),
    compiler_params=pltpu.CompilerParams(needs_layout_passes=False),
    out_type=[
        jax.ShapeDtypeStruct((_NPAD * 2, 128), jnp.float32),
        jax.ShapeDtypeStruct((2 * _NPAD,), jnp.float32),
        jax.ShapeDtypeStruct((_EP,), jnp.float32),
        jax.ShapeDtypeStruct((_NPAD * 2, 128), jnp.float32),
        jax.ShapeDtypeStruct((2 * _NPAD,), jnp.float32),
        jax.ShapeDtypeStruct((_EP,), jnp.float32),
    ],
    scratch_types=[
        pltpu.VMEM((_CSZ,), jnp.int32),
        pltpu.VMEM((_CSZ,), jnp.float32),
        pltpu.VMEM_SHARED((_NPAD,), jnp.float32),
        pltpu.VMEM_SHARED((_NPAD,), jnp.float32),
        pltpu.VMEM_SHARED((_NPAD,), jnp.float32),
        pltpu.VMEM_SHARED((_CH * 2, 128), jnp.float32),
        pltpu.VMEM((_B,), jnp.int32),
        pltpu.VMEM((_B,), jnp.int32),
        pltpu.VMEM((_B,), jnp.int32),
        pltpu.VMEM((_B,), jnp.float32),
        pltpu.VMEM((_B,), jnp.float32),
        pltpu.VMEM((_B,), jnp.float32),
        pltpu.VMEM((_B,), jnp.int32),
        pltpu.VMEM((2 * _B, 128), jnp.float32),
        pltpu.SemaphoreType.DMA,
    ],
)
def _sc_edge(sd1, asv1, adv1, hlo1, hhi1, sd2, asv2, adv2, hlo2, hhi2,
             zrows, zflat, out1, den1, ex1, out2, den2, ex2, *rest):
    _sc_conv(sd1, asv1, adv1, hlo1, hhi1, zrows, zflat, out1, den1, ex1,
             *rest)
    plsc.subcore_barrier()
    _sc_conv(sd2, asv2, adv2, hlo2, hhi2, zrows, zflat, out2, den2, ex2,
             *rest)


def _finish_body(unsc_ref, den_ref, o_ref):
    den = den_ref[...]
    rec = 1.0 / (den[:, 0] + den[:, 1] + 1e-16)
    o_ref[...] = jnp.maximum(unsc_ref[...] * rec[:, None], 0.0)


def _finish(unsc, den_t, row_off, rows):
    off_b = row_off // _BLK
    return pl.pallas_call(
        _finish_body,
        grid=(rows // _BLK,),
        in_specs=[
            pl.BlockSpec((_BLK, _D), lambda i: (i + off_b, 0)),
            pl.BlockSpec((_BLK, 2), lambda i: (i + off_b, 0)),
        ],
        out_specs=pl.BlockSpec((_BLK, _D), lambda i: (i, 0)),
        out_shape=jax.ShapeDtypeStruct((rows, _D), jnp.float32),
    )(unsc, den_t)


def kernel(edge_index, paper_edge_index, author_edge_index, x_s, x_t,
           Wls, bls, Wlt, blt, W1, a1s, a1d, W2, a2s, a2d):
    n_s = x_s.shape[0]
    n = n_s + x_t.shape[0]

    xcat = jnp.concatenate([x_s, x_t], axis=0)
    wl_st = jnp.stack([Wls, Wlt])
    bl_st = jnp.stack([bls[None, :], blt[None, :]])
    at = jnp.stack([a1s, a1d, a2s, a2d], axis=1)

    h1, h2, a = _dense_stage(xcat, wl_st, bl_st, W1, W2, at, n_s)

    src = edge_index[0]
    dst = edge_index[1] + n_s
    loops = jnp.arange(n, dtype=jnp.int32)

    def pack_edges(e_src, e_dst):
        npad = _EP - e_src.shape[0]
        pad_s = jnp.zeros((npad,), jnp.int32)
        pad_d = jnp.full((npad,), _N, jnp.int32)
        es = jnp.concatenate([e_src.astype(jnp.int32), pad_s])
        ed = jnp.concatenate([e_dst.astype(jnp.int32), pad_d])
        return jnp.bitwise_or(es, ed << 16)

    def pad_scores(v):
        return jnp.concatenate([v, jnp.zeros((_NPAD - n,), jnp.float32)])

    sd1 = pack_edges(jnp.concatenate([src, author_edge_index[0], loops]),
                     jnp.concatenate([dst, author_edge_index[1], loops]))
    sd2 = pack_edges(jnp.concatenate([dst, paper_edge_index[0], loops]),
                     jnp.concatenate([src, paper_edge_index[1], loops]))

    zrows = jnp.zeros((_CH // 8, 128), jnp.float32)
    zflat = jnp.zeros((_NPAD // 16,), jnp.float32)
    out1, den1, _, out2, den2, _ = _sc_edge(
        sd1, pad_scores(a[:, 0]), pad_scores(a[:, 1]),
        h1[:, :128], h1[:, 128:],
        sd2, pad_scores(a[:, 2]), pad_scores(a[:, 3]),
        h2[:, :128], h2[:, 128:], zrows, zflat)

    new_x_s = _finish(out2.reshape(_NPAD, _D), den2.reshape(2, _NPAD).T,
                      0, n_s)
    new_x_t = _finish(out1.reshape(_NPAD, _D), den1.reshape(2, _NPAD).T,
                      n_s, n - n_s)
    return (new_x_s, new_x_t)


# speedup vs baseline: 2.3712x; 1.0032x over previous
import functools

import jax
import jax.numpy as jnp
from jax import lax
from jax.experimental import pallas as pl
from jax.experimental.pallas import tpu as pltpu
from jax.experimental.pallas import tpu_sc as plsc

_D = 256
_BLK = 1000
_N = 20000
_NPAD = 20480
_EP = 340992
_EPT = _EP // 16
_CH = 5120
_CSZ = 576
_B = 96
_NG = _CSZ // _B
_NCH = _EPT // _CSZ
_RPT = _NPAD // 16


def _dense_body(x_ref, wl_ref, bl_ref, w1_ref, w2_ref, at_ref,
                h1_ref, h2_ref, a_ref):
    x = x_ref[...]
    t = jnp.dot(x, wl_ref[0], preferred_element_type=jnp.float32) + bl_ref[0]
    h1 = jnp.dot(t, w1_ref[...], preferred_element_type=jnp.float32)
    h2 = jnp.dot(t, w2_ref[...], preferred_element_type=jnp.float32)
    h1_ref[...] = h1
    h2_ref[...] = h2
    at = at_ref[...]
    a12 = jnp.dot(h1, at[:, :2], preferred_element_type=jnp.float32)
    a34 = jnp.dot(h2, at[:, 2:], preferred_element_type=jnp.float32)
    a_ref[...] = jnp.concatenate([a12, a34], axis=1)


def _dense_stage(xcat, wl_st, bl_st, w1, w2, at, n_s):
    n = xcat.shape[0]
    nbs = n_s // _BLK
    return pl.pallas_call(
        _dense_body,
        grid=(n // _BLK,),
        in_specs=[
            pl.BlockSpec((_BLK, _D), lambda i: (i, 0)),
            pl.BlockSpec((1, _D, _D), lambda i: (i // nbs, 0, 0)),
            pl.BlockSpec((1, 1, _D), lambda i: (i // nbs, 0, 0)),
            pl.BlockSpec((_D, _D), lambda i: (0, 0)),
            pl.BlockSpec((_D, _D), lambda i: (0, 0)),
            pl.BlockSpec((_D, 4), lambda i: (0, 0)),
        ],
        out_specs=[
            pl.BlockSpec((_BLK, _D), lambda i: (i, 0)),
            pl.BlockSpec((_BLK, _D), lambda i: (i, 0)),
            pl.BlockSpec((_BLK, 4), lambda i: (i, 0)),
        ],
        out_shape=[
            jax.ShapeDtypeStruct((n, _D), jnp.float32),
            jax.ShapeDtypeStruct((n, _D), jnp.float32),
            jax.ShapeDtypeStruct((n, 4), jnp.float32),
        ],
    )(xcat, wl_st, bl_st, w1, w2, at)


def _sc_conv(sd_hbm, asv_hbm, adv_hbm, hlo_hbm, hhi_hbm, zrows_hbm, zflat_hbm,
             out_hbm, den_hbm, ex_hbm,
             sdbuf, exbuf, asv_sh, adv_sh, den_sh, acc,
             sidx, didx, gidx, exb, asg, adg, sjdx, rbt, sem):
    c = lax.axis_index("c")
    s = lax.axis_index("s")

    sl = pl.ds(s * _RPT, _RPT)
    pltpu.sync_copy(asv_hbm.at[sl], asv_sh.at[sl])
    pltpu.sync_copy(adv_hbm.at[sl], adv_sh.at[sl])
    pltpu.sync_copy(zflat_hbm, den_sh.at[sl])
    plsc.subcore_barrier()

    def _chunk1(ci, carry):
        eoff = s * _EPT + ci * _CSZ
        pltpu.sync_copy(sd_hbm.at[pl.ds(eoff, _CSZ)], sdbuf)
        for v in range(_NG):
            goff = v * _B
            for w in range(_B // 16):
                sd16 = sdbuf[pl.ds(goff + w * 16, 16)]
                sidx[pl.ds(w * 16, 16)] = jnp.bitwise_and(sd16, 0xFFFF)
                didx[pl.ds(w * 16, 16)] = lax.shift_right_logical(sd16, 16)
            cp_a = pltpu.async_copy(asv_sh.at[sidx], asg, sem)
            cp_b = pltpu.async_copy(adv_sh.at[didx], adg, sem)
            cp_a.wait()
            cp_b.wait()
            for w in range(_B // 16):
                e = asg[pl.ds(w * 16, 16)] + adg[pl.ds(w * 16, 16)]
                e = jnp.where(e < 0.0, e * 0.2, e)
                exbuf[pl.ds(goff + w * 16, 16)] = jnp.exp(e)

            @pl.when(c == (v & 1))
            def _den():
                pltpu.sync_copy(exbuf.at[pl.ds(goff, _B)],
                                den_sh.at[didx], add=True)
        pltpu.sync_copy(exbuf, ex_hbm.at[pl.ds(eoff, _CSZ)])
        return carry
    lax.fori_loop(0, _NCH, _chunk1, 0)

    plsc.subcore_barrier()

    pltpu.sync_copy(den_sh.at[sl],
                    den_hbm.at[pl.ds(c * _NPAD + s * _RPT, _RPT)])

    def _pass(base):
        pltpu.sync_copy(zrows_hbm, acc.at[pl.ds(s * (_CH // 8), _CH // 8)])
        plsc.subcore_barrier()

        def _chunk2(ci, carry):
            eoff = s * _EPT + ci * _CSZ
            pltpu.sync_copy(sd_hbm.at[pl.ds(eoff, _CSZ)], sdbuf)
            pltpu.sync_copy(ex_hbm.at[pl.ds(eoff, _CSZ)], exbuf)
            for v in range(_NG):
                goff = v * _B
                for w in range(_B // 16):
                    sd16 = sdbuf[pl.ds(goff + w * 16, 16)]
                    s16 = jnp.bitwise_and(sd16, 0xFFFF)
                    d16 = lax.shift_right_logical(sd16, 16)
                    lo = d16 - base
                    inr = jnp.logical_and(lo >= 0, lo < _CH)
                    gidx[pl.ds(w * 16, 16)] = s16
                    sidx[pl.ds(w * 16, 16)] = jnp.where(inr, lo, 0)
                    ex16 = exbuf[pl.ds(goff + w * 16, 16)]
                    exb[pl.ds(w * 16, 16)] = jnp.where(inr, ex16, 0.0)
                cp_lo = pltpu.async_copy(hlo_hbm.at[gidx],
                                         rbt.at[pl.ds(0, _B)], sem)
                cp_hi = pltpu.async_copy(hhi_hbm.at[gidx],
                                         rbt.at[pl.ds(_B, _B)], sem)
                cp_lo.wait()
                cp_hi.wait()

                ci16 = lax.iota(jnp.int32, 16)

                def _scale(r2, cc):
                    for rr in range(3):
                        ridx = jnp.full((16,), r2 * 3 + rr, jnp.int32)
                        ex_s = plsc.load_gather(exb, [ridx])
                        for j in range(16):
                            rj = ridx + ((j // 8) * _B)
                            cidx = ci16 + ((j % 8) * 16)
                            v2 = plsc.load_gather(rbt, [rj, cidx])
                            plsc.store_scatter(rbt, [rj, cidx], v2 * ex_s)
                    return cc
                lax.fori_loop(0, _B // 3, _scale, 0)

                for w in range(_B // 16):
                    lo16 = sidx[pl.ds(w * 16, 16)]
                    sjdx[pl.ds(w * 16, 16)] = lo16 * 2
                    didx[pl.ds(w * 16, 16)] = lo16 * 2 + 1
                cp0 = pltpu.async_copy(rbt.at[pl.ds(0, _B)],
                                       acc.at[sjdx], sem, add=True)
                cp1 = pltpu.async_copy(rbt.at[pl.ds(_B, _B)],
                                       acc.at[didx], sem, add=True)
                cp0.wait()
                cp1.wait()
            return carry
        lax.fori_loop(0, _NCH, _chunk2, 0)
        plsc.subcore_barrier()

        d0 = s * (_CH // 16)
        for o in range(0, _CH // 16, 64):
            pltpu.sync_copy(acc.at[pl.ds((d0 + o) * 2, 64 * 2)],
                            out_hbm.at[pl.ds((base + d0 + o) * 2, 64 * 2)])
        plsc.subcore_barrier()

    _pass((2 * c + 0) * _CH)
    _pass((2 * c + 1) * _CH)


@functools.partial(
    pl.kernel,
    mesh=plsc.VectorSubcoreMesh(core_axis_name="c", subcore_axis_name="s"),
    compiler_params=pltpu.CompilerParams(needs_layout_passes=False),
    out_type=[
        jax.ShapeDtypeStruct((_NPAD * 2, 128), jnp.float32),
        jax.ShapeDtypeStruct((2 * _NPAD,), jnp.float32),
        jax.ShapeDtypeStruct((_EP,), jnp.float32),
        jax.ShapeDtypeStruct((_NPAD * 2, 128), jnp.float32),
        jax.ShapeDtypeStruct((2 * _NPAD,), jnp.float32),
        jax.ShapeDtypeStruct((_EP,), jnp.float32),
    ],
    scratch_types=[
        pltpu.VMEM((_CSZ,), jnp.int32),
        pltpu.VMEM((_CSZ,), jnp.float32),
        pltpu.VMEM_SHARED((_NPAD,), jnp.float32),
        pltpu.VMEM_SHARED((_NPAD,), jnp.float32),
        pltpu.VMEM_SHARED((_NPAD,), jnp.float32),
        pltpu.VMEM_SHARED((_CH * 2, 128), jnp.float32),
        pltpu.VMEM((_B,), jnp.int32),
        pltpu.VMEM((_B,), jnp.int32),
        pltpu.VMEM((_B,), jnp.int32),
        pltpu.VMEM((_B,), jnp.float32),
        pltpu.VMEM((_B,), jnp.float32),
        pltpu.VMEM((_B,), jnp.float32),
        pltpu.VMEM((_B,), jnp.int32),
        pltpu.VMEM((2 * _B, 128), jnp.float32),
        pltpu.SemaphoreType.DMA,
    ],
)
def _sc_edge(sd1, asv1, adv1, hlo1, hhi1, sd2, asv2, adv2, hlo2, hhi2,
             zrows, zflat, out1, den1, ex1, out2, den2, ex2, *rest):
    _sc_conv(sd1, asv1, adv1, hlo1, hhi1, zrows, zflat, out1, den1, ex1,
             *rest)
    plsc.subcore_barrier()
    _sc_conv(sd2, asv2, adv2, hlo2, hhi2, zrows, zflat, out2, den2, ex2,
             *rest)


def _finish_body(unsc_ref, den_ref, o_ref):
    den = den_ref[...]
    rec = 1.0 / (den[:, 0] + den[:, 1] + 1e-16)
    o_ref[...] = jnp.maximum(unsc_ref[...] * rec[:, None], 0.0)


def _finish(unsc, den_t, row_off, rows):
    off_b = row_off // _BLK
    return pl.pallas_call(
        _finish_body,
        grid=(rows // _BLK,),
        in_specs=[
            pl.BlockSpec((_BLK, _D), lambda i: (i + off_b, 0)),
            pl.BlockSpec((_BLK, 2), lambda i: (i + off_b, 0)),
        ],
        out_specs=pl.BlockSpec((_BLK, _D), lambda i: (i, 0)),
        out_shape=jax.ShapeDtypeStruct((rows, _D), jnp.float32),
    )(unsc, den_t)


def kernel(edge_index, paper_edge_index, author_edge_index, x_s, x_t,
           Wls, bls, Wlt, blt, W1, a1s, a1d, W2, a2s, a2d):
    n_s = x_s.shape[0]
    n = n_s + x_t.shape[0]

    xcat = jnp.concatenate([x_s, x_t], axis=0)
    wl_st = jnp.stack([Wls, Wlt])
    bl_st = jnp.stack([bls[None, :], blt[None, :]])
    at = jnp.stack([a1s, a1d, a2s, a2d], axis=1)

    h1, h2, a = _dense_stage(xcat, wl_st, bl_st, W1, W2, at, n_s)

    src = edge_index[0]
    dst = edge_index[1] + n_s
    loops = jnp.arange(n, dtype=jnp.int32)

    def pack_edges(e_src, e_dst):
        npad = _EP - e_src.shape[0]
        pad_s = jnp.zeros((npad,), jnp.int32)
        pad_d = jnp.full((npad,), _N, jnp.int32)
        es = jnp.concatenate([e_src.astype(jnp.int32), pad_s])
        ed = jnp.concatenate([e_dst.astype(jnp.int32), pad_d])
        return jnp.bitwise_or(es, ed << 16)

    def pad_scores(v):
        return jnp.concatenate([v, jnp.zeros((_NPAD - n,), jnp.float32)])

    sd1 = pack_edges(jnp.concatenate([src, author_edge_index[0], loops]),
                     jnp.concatenate([dst, author_edge_index[1], loops]))
    sd2 = pack_edges(jnp.concatenate([dst, paper_edge_index[0], loops]),
                     jnp.concatenate([src, paper_edge_index[1], loops]))

    zrows = jnp.zeros((_CH // 8, 128), jnp.float32)
    zflat = jnp.zeros((_NPAD // 16,), jnp.float32)
    out1, den1, _, out2, den2, _ = _sc_edge(
        sd1, pad_scores(a[:, 0]), pad_scores(a[:, 1]),
        h1[:, :128], h1[:, 128:],
        sd2, pad_scores(a[:, 2]), pad_scores(a[:, 3]),
        h2[:, :128], h2[:, 128:], zrows, zflat)

    new_x_s = _finish(out2.reshape(_NPAD, _D), den2.reshape(2, _NPAD).T,
                      0, n_s)
    new_x_t = _finish(out1.reshape(_NPAD, _D), den1.reshape(2, _NPAD).T,
                      n_s, n - n_s)
    return (new_x_s, new_x_t)


# speedup vs baseline: 2.7463x; 1.1582x over previous
import functools

import jax
import jax.numpy as jnp
from jax import lax
from jax.experimental import pallas as pl
from jax.experimental.pallas import tpu as pltpu
from jax.experimental.pallas import tpu_sc as plsc

_D = 256
_BLK = 1000
_N = 20000
_NPAD = 20480
_EP = 340992
_EPT = _EP // 16
_CH = 5120
_CSZ = 576
_B = 48
_DEN_B = 96
_NG = _CSZ // _B
_NCH = _EPT // _CSZ
_RPT = _NPAD // 16


def _dense_body(x_ref, wl_ref, bl_ref, w1_ref, w2_ref, at_ref,
                h1_ref, h2_ref, a_ref):
    x = x_ref[...]
    t = jnp.dot(x, wl_ref[0], preferred_element_type=jnp.float32) + bl_ref[0]
    h1 = jnp.dot(t, w1_ref[...], preferred_element_type=jnp.float32)
    h2 = jnp.dot(t, w2_ref[...], preferred_element_type=jnp.float32)
    h1_ref[...] = h1
    h2_ref[...] = h2
    at = at_ref[...]
    a12 = jnp.dot(h1, at[:, :2], preferred_element_type=jnp.float32)
    a34 = jnp.dot(h2, at[:, 2:], preferred_element_type=jnp.float32)
    a_ref[...] = jnp.concatenate([a12, a34], axis=1)


def _dense_stage(xcat, wl_st, bl_st, w1, w2, at, n_s):
    n = xcat.shape[0]
    nbs = n_s // _BLK
    return pl.pallas_call(
        _dense_body,
        grid=(n // _BLK,),
        in_specs=[
            pl.BlockSpec((_BLK, _D), lambda i: (i, 0)),
            pl.BlockSpec((1, _D, _D), lambda i: (i // nbs, 0, 0)),
            pl.BlockSpec((1, 1, _D), lambda i: (i // nbs, 0, 0)),
            pl.BlockSpec((_D, _D), lambda i: (0, 0)),
            pl.BlockSpec((_D, _D), lambda i: (0, 0)),
            pl.BlockSpec((_D, 4), lambda i: (0, 0)),
        ],
        out_specs=[
            pl.BlockSpec((_BLK, _D), lambda i: (i, 0)),
            pl.BlockSpec((_BLK, _D), lambda i: (i, 0)),
            pl.BlockSpec((_BLK, 4), lambda i: (i, 0)),
        ],
        out_shape=[
            jax.ShapeDtypeStruct((n, _D), jnp.float32),
            jax.ShapeDtypeStruct((n, _D), jnp.float32),
            jax.ShapeDtypeStruct((n, 4), jnp.float32),
        ],
    )(xcat, wl_st, bl_st, w1, w2, at)


def _sc_conv(sd_hbm, asv_hbm, adv_hbm, hlo_hbm, hhi_hbm, zrows_hbm, zflat_hbm,
             out_hbm, den_hbm, ex_hbm,
             sdbuf, exbuf, asv_sh, adv_sh, den_sh, acc,
             sidx, didx, asg, adg,
             gA, gB, eA, eB, jloA, jloB, jhiA, jhiB,
             rbt, gsA, gsB, ssA, ssB):
    c = lax.axis_index("c")
    s = lax.axis_index("s")

    sl = pl.ds(s * _RPT, _RPT)
    pltpu.sync_copy(asv_hbm.at[sl], asv_sh.at[sl])
    pltpu.sync_copy(adv_hbm.at[sl], adv_sh.at[sl])
    pltpu.sync_copy(zflat_hbm, den_sh.at[sl])
    plsc.subcore_barrier()

    def _chunk1(ci, carry):
        eoff = s * _EPT + ci * _CSZ
        pltpu.sync_copy(sd_hbm.at[pl.ds(eoff, _CSZ)], sdbuf)
        for v in range(_CSZ // _DEN_B):
            goff = v * _DEN_B
            for w in range(_DEN_B // 16):
                sd16 = sdbuf[pl.ds(goff + w * 16, 16)]
                sidx[pl.ds(w * 16, 16)] = jnp.bitwise_and(sd16, 0xFFFF)
                didx[pl.ds(w * 16, 16)] = lax.shift_right_logical(sd16, 16)
            cp_a = pltpu.async_copy(asv_sh.at[sidx], asg, gsA)
            cp_b = pltpu.async_copy(adv_sh.at[didx], adg, gsB)
            cp_a.wait()
            cp_b.wait()
            for w in range(_DEN_B // 16):
                e = asg[pl.ds(w * 16, 16)] + adg[pl.ds(w * 16, 16)]
                e = jnp.where(e < 0.0, e * 0.2, e)
                exbuf[pl.ds(goff + w * 16, 16)] = jnp.exp(e)

            @pl.when(c == (v & 1))
            def _den():
                pltpu.sync_copy(exbuf.at[pl.ds(goff, _DEN_B)],
                                den_sh.at[didx], add=True)
        pltpu.sync_copy(exbuf, ex_hbm.at[pl.ds(eoff, _CSZ)])
        return carry
    lax.fori_loop(0, _NCH, _chunk1, 0)

    plsc.subcore_barrier()

    pltpu.sync_copy(den_sh.at[sl],
                    den_hbm.at[pl.ds(c * _NPAD + s * _RPT, _RPT)])

    def _pass(base):
        pltpu.sync_copy(zrows_hbm, acc.at[pl.ds(s * (_CH // 8), _CH // 8)])
        plsc.subcore_barrier()

        ci16 = lax.iota(jnp.int32, 16)
        sets = ((gA, eA, jloA, jhiA, gsA, ssA),
                (gB, eB, jloB, jhiB, gsB, ssB))

        def _prep(v, g, e, jlo, jhi):
            goff = v * _B
            for w in range(_B // 16):
                sd16 = sdbuf[pl.ds(goff + w * 16, 16)]
                s16 = jnp.bitwise_and(sd16, 0xFFFF)
                d16 = lax.shift_right_logical(sd16, 16)
                lo = d16 - base
                inr = jnp.logical_and(lo >= 0, lo < _CH)
                loc = jnp.where(inr, lo, 0)
                g[pl.ds(w * 16, 16)] = s16
                jlo[pl.ds(w * 16, 16)] = loc * 2
                jhi[pl.ds(w * 16, 16)] = loc * 2 + 1
                ex16 = exbuf[pl.ds(goff + w * 16, 16)]
                e[pl.ds(w * 16, 16)] = jnp.where(inr, ex16, 0.0)

        def _scale(p, e):
            def body(r2, cc):
                for rr in range(3):
                    ridx = jnp.full((16,), r2 * 3 + rr, jnp.int32)
                    ex_s = plsc.load_gather(e, [ridx])
                    for j in range(16):
                        rj = ridx + (p * 2 * _B + (j // 8) * _B)
                        cidx = ci16 + ((j % 8) * 16)
                        v2 = plsc.load_gather(rbt, [rj, cidx])
                        plsc.store_scatter(rbt, [rj, cidx], v2 * ex_s)
                return cc
            lax.fori_loop(0, _B // 3, body, 0)

        def _lo_slice(p):
            return rbt.at[pl.ds(p * 2 * _B, _B)]

        def _hi_slice(p):
            return rbt.at[pl.ds(p * 2 * _B + _B, _B)]

        def _wait_scatter(p):
            _, _, jlo, jhi, _, ssem = sets[p]
            pltpu.make_async_copy(_lo_slice(p), acc.at[jlo], ssem).wait()
            pltpu.make_async_copy(_hi_slice(p), acc.at[jhi], ssem).wait()

        def _chunk2(ci, carry):
            eoff = s * _EPT + ci * _CSZ
            pltpu.sync_copy(sd_hbm.at[pl.ds(eoff, _CSZ)], sdbuf)
            pltpu.sync_copy(ex_hbm.at[pl.ds(eoff, _CSZ)], exbuf)

            def _pair(kk, cc):
                gathers = []
                for p in (0, 1):
                    g, e, jlo, jhi, gsem, _ = sets[p]

                    @pl.when(jnp.logical_or(kk > 0, ci > 0))
                    def _():
                        _wait_scatter(p)
                    _prep(kk * 2 + p, g, e, jlo, jhi)
                    cl = pltpu.async_copy(hlo_hbm.at[g], _lo_slice(p), gsem)
                    ch = pltpu.async_copy(hhi_hbm.at[g], _hi_slice(p), gsem)
                    gathers.append((cl, ch))
                for p in (0, 1):
                    _, e, jlo, jhi, _, ssem = sets[p]
                    gathers[p][0].wait()
                    gathers[p][1].wait()
                    _scale(p, e)
                    pltpu.async_copy(_lo_slice(p), acc.at[jlo], ssem,
                                     add=True)
                    pltpu.async_copy(_hi_slice(p), acc.at[jhi], ssem,
                                     add=True)
                return cc
            lax.fori_loop(0, _NG // 2, _pair, 0)
            return carry
        lax.fori_loop(0, _NCH, _chunk2, 0)
        _wait_scatter(0)
        _wait_scatter(1)
        plsc.subcore_barrier()

        d0 = s * (_CH // 16)
        for o in range(0, _CH // 16, 64):
            pltpu.sync_copy(acc.at[pl.ds((d0 + o) * 2, 64 * 2)],
                            out_hbm.at[pl.ds((base + d0 + o) * 2, 64 * 2)])
        plsc.subcore_barrier()

    _pass((2 * c + 0) * _CH)
    _pass((2 * c + 1) * _CH)


@functools.partial(
    pl.kernel,
    mesh=plsc.VectorSubcoreMesh(core_axis_name="c", subcore_axis_name="s"),
    compiler_params=pltpu.CompilerParams(needs_layout_passes=False),
    out_type=[
        jax.ShapeDtypeStruct((_NPAD * 2, 128), jnp.float32),
        jax.ShapeDtypeStruct((2 * _NPAD,), jnp.float32),
        jax.ShapeDtypeStruct((_EP,), jnp.float32),
        jax.ShapeDtypeStruct((_NPAD * 2, 128), jnp.float32),
        jax.ShapeDtypeStruct((2 * _NPAD,), jnp.float32),
        jax.ShapeDtypeStruct((_EP,), jnp.float32),
    ],
    scratch_types=[
        pltpu.VMEM((_CSZ,), jnp.int32),
        pltpu.VMEM((_CSZ,), jnp.float32),
        pltpu.VMEM_SHARED((_NPAD,), jnp.float32),
        pltpu.VMEM_SHARED((_NPAD,), jnp.float32),
        pltpu.VMEM_SHARED((_NPAD,), jnp.float32),
        pltpu.VMEM_SHARED((_CH * 2, 128), jnp.float32),
        pltpu.VMEM((_DEN_B,), jnp.int32),
        pltpu.VMEM((_DEN_B,), jnp.int32),
        pltpu.VMEM((_DEN_B,), jnp.float32),
        pltpu.VMEM((_DEN_B,), jnp.float32),
        pltpu.VMEM((_B,), jnp.int32),
        pltpu.VMEM((_B,), jnp.int32),
        pltpu.VMEM((_B,), jnp.float32),
        pltpu.VMEM((_B,), jnp.float32),
        pltpu.VMEM((_B,), jnp.int32),
        pltpu.VMEM((_B,), jnp.int32),
        pltpu.VMEM((_B,), jnp.int32),
        pltpu.VMEM((_B,), jnp.int32),
        pltpu.VMEM((4 * _B, 128), jnp.float32),
        pltpu.SemaphoreType.DMA,
        pltpu.SemaphoreType.DMA,
        pltpu.SemaphoreType.DMA,
        pltpu.SemaphoreType.DMA,
    ],
)
def _sc_edge(sd1, asv1, adv1, hlo1, hhi1, sd2, asv2, adv2, hlo2, hhi2,
             zrows, zflat, out1, den1, ex1, out2, den2, ex2, *rest):
    _sc_conv(sd1, asv1, adv1, hlo1, hhi1, zrows, zflat, out1, den1, ex1,
             *rest)
    plsc.subcore_barrier()
    _sc_conv(sd2, asv2, adv2, hlo2, hhi2, zrows, zflat, out2, den2, ex2,
             *rest)


def _finish_body(unsc_ref, den_ref, o_ref):
    den = den_ref[...]
    rec = 1.0 / (den[:, 0] + den[:, 1] + 1e-16)
    o_ref[...] = jnp.maximum(unsc_ref[...] * rec[:, None], 0.0)


def _finish(unsc, den_t, row_off, rows):
    off_b = row_off // _BLK
    return pl.pallas_call(
        _finish_body,
        grid=(rows // _BLK,),
        in_specs=[
            pl.BlockSpec((_BLK, _D), lambda i: (i + off_b, 0)),
            pl.BlockSpec((_BLK, 2), lambda i: (i + off_b, 0)),
        ],
        out_specs=pl.BlockSpec((_BLK, _D), lambda i: (i, 0)),
        out_shape=jax.ShapeDtypeStruct((rows, _D), jnp.float32),
    )(unsc, den_t)


def kernel(edge_index, paper_edge_index, author_edge_index, x_s, x_t,
           Wls, bls, Wlt, blt, W1, a1s, a1d, W2, a2s, a2d):
    n_s = x_s.shape[0]
    n = n_s + x_t.shape[0]

    xcat = jnp.concatenate([x_s, x_t], axis=0)
    wl_st = jnp.stack([Wls, Wlt])
    bl_st = jnp.stack([bls[None, :], blt[None, :]])
    at = jnp.stack([a1s, a1d, a2s, a2d], axis=1)

    h1, h2, a = _dense_stage(xcat, wl_st, bl_st, W1, W2, at, n_s)

    src = edge_index[0]
    dst = edge_index[1] + n_s
    loops = jnp.arange(n, dtype=jnp.int32)

    def pack_edges(e_src, e_dst):
        npad = _EP - e_src.shape[0]
        pad_s = jnp.zeros((npad,), jnp.int32)
        pad_d = jnp.full((npad,), _N, jnp.int32)
        es = jnp.concatenate([e_src.astype(jnp.int32), pad_s])
        ed = jnp.concatenate([e_dst.astype(jnp.int32), pad_d])
        return jnp.bitwise_or(es, ed << 16)

    def pad_scores(v):
        return jnp.concatenate([v, jnp.zeros((_NPAD - n,), jnp.float32)])

    sd1 = pack_edges(jnp.concatenate([src, author_edge_index[0], loops]),
                     jnp.concatenate([dst, author_edge_index[1], loops]))
    sd2 = pack_edges(jnp.concatenate([dst, paper_edge_index[0], loops]),
                     jnp.concatenate([src, paper_edge_index[1], loops]))

    zrows = jnp.zeros((_CH // 8, 128), jnp.float32)
    zflat = jnp.zeros((_NPAD // 16,), jnp.float32)
    out1, den1, _, out2, den2, _ = _sc_edge(
        sd1, pad_scores(a[:, 0]), pad_scores(a[:, 1]),
        h1[:, :128], h1[:, 128:],
        sd2, pad_scores(a[:, 2]), pad_scores(a[:, 3]),
        h2[:, :128], h2[:, 128:], zrows, zflat)

    new_x_s = _finish(out2.reshape(_NPAD, _D), den2.reshape(2, _NPAD).T,
                      0, n_s)
    new_x_t = _finish(out1.reshape(_NPAD, _D), den1.reshape(2, _NPAD).T,
                      n_s, n - n_s)
    return (new_x_s, new_x_t)


# speedup vs baseline: 2.7494x; 1.0011x over previous
import functools

import jax
import jax.numpy as jnp
from jax import lax
from jax.experimental import pallas as pl
from jax.experimental.pallas import tpu as pltpu
from jax.experimental.pallas import tpu_sc as plsc

_D = 256
_BLK = 1000
_N = 20000
_NPAD = 20480
_EP = 340992
_EPT = _EP // 16
_CH = 5120
_CSZ = 576
_B = 48
_DEN_B = 96
_NG = _CSZ // _B
_NCH = _EPT // _CSZ
_RPT = _NPAD // 16


def _dense_body(x_ref, wl_ref, bl_ref, w1_ref, w2_ref, at_ref,
                h1_ref, h2_ref, a_ref):
    x = x_ref[...]
    t = jnp.dot(x, wl_ref[0], preferred_element_type=jnp.float32) + bl_ref[0]
    h1 = jnp.dot(t, w1_ref[...], preferred_element_type=jnp.float32)
    h2 = jnp.dot(t, w2_ref[...], preferred_element_type=jnp.float32)
    h1_ref[...] = h1
    h2_ref[...] = h2
    at = at_ref[...]
    a12 = jnp.dot(h1, at[:, :2], preferred_element_type=jnp.float32)
    a34 = jnp.dot(h2, at[:, 2:], preferred_element_type=jnp.float32)
    a_ref[...] = jnp.concatenate([a12, a34], axis=1)


def _dense_stage(xcat, wl_st, bl_st, w1, w2, at, n_s):
    n = xcat.shape[0]
    nbs = n_s // _BLK
    return pl.pallas_call(
        _dense_body,
        grid=(n // _BLK,),
        in_specs=[
            pl.BlockSpec((_BLK, _D), lambda i: (i, 0)),
            pl.BlockSpec((1, _D, _D), lambda i: (i // nbs, 0, 0)),
            pl.BlockSpec((1, 1, _D), lambda i: (i // nbs, 0, 0)),
            pl.BlockSpec((_D, _D), lambda i: (0, 0)),
            pl.BlockSpec((_D, _D), lambda i: (0, 0)),
            pl.BlockSpec((_D, 4), lambda i: (0, 0)),
        ],
        out_specs=[
            pl.BlockSpec((_BLK, _D), lambda i: (i, 0)),
            pl.BlockSpec((_BLK, _D), lambda i: (i, 0)),
            pl.BlockSpec((_BLK, 4), lambda i: (i, 0)),
        ],
        out_shape=[
            jax.ShapeDtypeStruct((n, _D), jnp.float32),
            jax.ShapeDtypeStruct((n, _D), jnp.float32),
            jax.ShapeDtypeStruct((n, 4), jnp.float32),
        ],
    )(xcat, wl_st, bl_st, w1, w2, at)


def _sc_conv(sd_hbm, asv_hbm, adv_hbm, hlo_hbm, hhi_hbm, zrows_hbm, zflat_hbm,
             out_hbm, den_hbm, ex_hbm,
             sdbuf, exbuf, asv_sh, adv_sh, den_sh, acc,
             sidx, didx, asg, adg, sidxB, didxB, asgB, adgB,
             gA, gB, eA, eB, jloA, jloB, jhiA, jhiB,
             rbt, gsA, gsB, ssA, ssB):
    c = lax.axis_index("c")
    s = lax.axis_index("s")

    sl = pl.ds(s * _RPT, _RPT)
    pltpu.sync_copy(asv_hbm.at[sl], asv_sh.at[sl])
    pltpu.sync_copy(adv_hbm.at[sl], adv_sh.at[sl])
    pltpu.sync_copy(zflat_hbm, den_sh.at[sl])
    plsc.subcore_barrier()

    dsets = ((sidx, didx, asg, adg, gsA), (sidxB, didxB, asgB, adgB, gsB))

    def _chunk1(ci, carry):
        eoff = s * _EPT + ci * _CSZ
        pltpu.sync_copy(sd_hbm.at[pl.ds(eoff, _CSZ)], sdbuf)

        def _dpair(u, cc):
            hnds = []
            for p in (0, 1):
                si, di, ag, dg, gsem = dsets[p]
                goff = (u * 2 + p) * _DEN_B
                for w in range(_DEN_B // 16):
                    sd16 = sdbuf[pl.ds(goff + w * 16, 16)]
                    si[pl.ds(w * 16, 16)] = jnp.bitwise_and(sd16, 0xFFFF)
                    di[pl.ds(w * 16, 16)] = lax.shift_right_logical(sd16, 16)
                hnds.append((pltpu.async_copy(asv_sh.at[si], ag, gsem),
                             pltpu.async_copy(adv_sh.at[di], dg, gsem)))
            for p in (0, 1):
                si, di, ag, dg, gsem = dsets[p]
                goff = (u * 2 + p) * _DEN_B
                hnds[p][0].wait()
                hnds[p][1].wait()
                for w in range(_DEN_B // 16):
                    e = ag[pl.ds(w * 16, 16)] + dg[pl.ds(w * 16, 16)]
                    e = jnp.where(e < 0.0, e * 0.2, e)
                    exbuf[pl.ds(goff + w * 16, 16)] = jnp.exp(e)

                @pl.when(c == p)
                def _den():
                    pltpu.sync_copy(exbuf.at[pl.ds(goff, _DEN_B)],
                                    den_sh.at[di], add=True)
            return cc
        lax.fori_loop(0, _CSZ // _DEN_B // 2, _dpair, 0)
        pltpu.sync_copy(exbuf, ex_hbm.at[pl.ds(eoff, _CSZ)])
        return carry
    lax.fori_loop(0, _NCH, _chunk1, 0)

    plsc.subcore_barrier()

    pltpu.sync_copy(den_sh.at[sl],
                    den_hbm.at[pl.ds(c * _NPAD + s * _RPT, _RPT)])

    def _pass(base):
        pltpu.sync_copy(zrows_hbm, acc.at[pl.ds(s * (_CH // 8), _CH // 8)])
        plsc.subcore_barrier()

        ci16 = lax.iota(jnp.int32, 16)
        sets = ((gA, eA, jloA, jhiA, gsA, ssA),
                (gB, eB, jloB, jhiB, gsB, ssB))

        def _prep(v, g, e, jlo, jhi):
            goff = v * _B
            for w in range(_B // 16):
                sd16 = sdbuf[pl.ds(goff + w * 16, 16)]
                s16 = jnp.bitwise_and(sd16, 0xFFFF)
                d16 = lax.shift_right_logical(sd16, 16)
                lo = d16 - base
                inr = jnp.logical_and(lo >= 0, lo < _CH)
                loc = jnp.where(inr, lo, 0)
                g[pl.ds(w * 16, 16)] = s16
                jlo[pl.ds(w * 16, 16)] = loc * 2
                jhi[pl.ds(w * 16, 16)] = loc * 2 + 1
                ex16 = exbuf[pl.ds(goff + w * 16, 16)]
                e[pl.ds(w * 16, 16)] = jnp.where(inr, ex16, 0.0)

        def _scale(p, e):
            def body(r2, cc):
                for rr in range(3):
                    ridx = jnp.full((16,), r2 * 3 + rr, jnp.int32)
                    ex_s = plsc.load_gather(e, [ridx])
                    for j in range(16):
                        rj = ridx + (p * 2 * _B + (j // 8) * _B)
                        cidx = ci16 + ((j % 8) * 16)
                        v2 = plsc.load_gather(rbt, [rj, cidx])
                        plsc.store_scatter(rbt, [rj, cidx], v2 * ex_s)
                return cc
            lax.fori_loop(0, _B // 3, body, 0)

        def _lo_slice(p):
            return rbt.at[pl.ds(p * 2 * _B, _B)]

        def _hi_slice(p):
            return rbt.at[pl.ds(p * 2 * _B + _B, _B)]

        def _wait_scatter(p):
            _, _, jlo, jhi, _, ssem = sets[p]
            pltpu.make_async_copy(_lo_slice(p), acc.at[jlo], ssem).wait()
            pltpu.make_async_copy(_hi_slice(p), acc.at[jhi], ssem).wait()

        def _chunk2(ci, carry):
            eoff = s * _EPT + ci * _CSZ
            pltpu.sync_copy(sd_hbm.at[pl.ds(eoff, _CSZ)], sdbuf)
            pltpu.sync_copy(ex_hbm.at[pl.ds(eoff, _CSZ)], exbuf)

            def _pair(kk, cc):
                gathers = []
                for p in (0, 1):
                    g, e, jlo, jhi, gsem, _ = sets[p]

                    @pl.when(jnp.logical_or(kk > 0, ci > 0))
                    def _():
                        _wait_scatter(p)
                    _prep(kk * 2 + p, g, e, jlo, jhi)
                    cl = pltpu.async_copy(hlo_hbm.at[g], _lo_slice(p), gsem)
                    ch = pltpu.async_copy(hhi_hbm.at[g], _hi_slice(p), gsem)
                    gathers.append((cl, ch))
                for p in (0, 1):
                    _, e, jlo, jhi, _, ssem = sets[p]
                    gathers[p][0].wait()
                    gathers[p][1].wait()
                    _scale(p, e)
                    pltpu.async_copy(_lo_slice(p), acc.at[jlo], ssem,
                                     add=True)
                    pltpu.async_copy(_hi_slice(p), acc.at[jhi], ssem,
                                     add=True)
                return cc
            lax.fori_loop(0, _NG // 2, _pair, 0)
            return carry
        lax.fori_loop(0, _NCH, _chunk2, 0)
        _wait_scatter(0)
        _wait_scatter(1)
        plsc.subcore_barrier()

        d0 = s * (_CH // 16)
        for o in range(0, _CH // 16, 64):
            pltpu.sync_copy(acc.at[pl.ds((d0 + o) * 2, 64 * 2)],
                            out_hbm.at[pl.ds((base + d0 + o) * 2, 64 * 2)])
        plsc.subcore_barrier()

    _pass((2 * c + 0) * _CH)
    _pass((2 * c + 1) * _CH)


@functools.partial(
    pl.kernel,
    mesh=plsc.VectorSubcoreMesh(core_axis_name="c", subcore_axis_name="s"),
    compiler_params=pltpu.CompilerParams(needs_layout_passes=False),
    out_type=[
        jax.ShapeDtypeStruct((_NPAD * 2, 128), jnp.float32),
        jax.ShapeDtypeStruct((2 * _NPAD,), jnp.float32),
        jax.ShapeDtypeStruct((_EP,), jnp.float32),
        jax.ShapeDtypeStruct((_NPAD * 2, 128), jnp.float32),
        jax.ShapeDtypeStruct((2 * _NPAD,), jnp.float32),
        jax.ShapeDtypeStruct((_EP,), jnp.float32),
    ],
    scratch_types=[
        pltpu.VMEM((_CSZ,), jnp.int32),
        pltpu.VMEM((_CSZ,), jnp.float32),
        pltpu.VMEM_SHARED((_NPAD,), jnp.float32),
        pltpu.VMEM_SHARED((_NPAD,), jnp.float32),
        pltpu.VMEM_SHARED((_NPAD,), jnp.float32),
        pltpu.VMEM_SHARED((_CH * 2, 128), jnp.float32),
        pltpu.VMEM((_DEN_B,), jnp.int32),
        pltpu.VMEM((_DEN_B,), jnp.int32),
        pltpu.VMEM((_DEN_B,), jnp.float32),
        pltpu.VMEM((_DEN_B,), jnp.float32),
        pltpu.VMEM((_DEN_B,), jnp.int32),
        pltpu.VMEM((_DEN_B,), jnp.int32),
        pltpu.VMEM((_DEN_B,), jnp.float32),
        pltpu.VMEM((_DEN_B,), jnp.float32),
        pltpu.VMEM((_B,), jnp.int32),
        pltpu.VMEM((_B,), jnp.int32),
        pltpu.VMEM((_B,), jnp.float32),
        pltpu.VMEM((_B,), jnp.float32),
        pltpu.VMEM((_B,), jnp.int32),
        pltpu.VMEM((_B,), jnp.int32),
        pltpu.VMEM((_B,), jnp.int32),
        pltpu.VMEM((_B,), jnp.int32),
        pltpu.VMEM((4 * _B, 128), jnp.float32),
        pltpu.SemaphoreType.DMA,
        pltpu.SemaphoreType.DMA,
        pltpu.SemaphoreType.DMA,
        pltpu.SemaphoreType.DMA,
    ],
)
def _sc_edge(sd1, asv1, adv1, hlo1, hhi1, sd2, asv2, adv2, hlo2, hhi2,
             zrows, zflat, out1, den1, ex1, out2, den2, ex2, *rest):
    _sc_conv(sd1, asv1, adv1, hlo1, hhi1, zrows, zflat, out1, den1, ex1,
             *rest)
    plsc.subcore_barrier()
    _sc_conv(sd2, asv2, adv2, hlo2, hhi2, zrows, zflat, out2, den2, ex2,
             *rest)


def _finish_body(unsc_ref, den_ref, o_ref):
    den = den_ref[...]
    rec = 1.0 / (den[:, 0] + den[:, 1] + 1e-16)
    o_ref[...] = jnp.maximum(unsc_ref[...] * rec[:, None], 0.0)


def _finish(unsc, den_t, row_off, rows):
    off_b = row_off // _BLK
    return pl.pallas_call(
        _finish_body,
        grid=(rows // _BLK,),
        in_specs=[
            pl.BlockSpec((_BLK, _D), lambda i: (i + off_b, 0)),
            pl.BlockSpec((_BLK, 2), lambda i: (i + off_b, 0)),
        ],
        out_specs=pl.BlockSpec((_BLK, _D), lambda i: (i, 0)),
        out_shape=jax.ShapeDtypeStruct((rows, _D), jnp.float32),
    )(unsc, den_t)


def kernel(edge_index, paper_edge_index, author_edge_index, x_s, x_t,
           Wls, bls, Wlt, blt, W1, a1s, a1d, W2, a2s, a2d):
    n_s = x_s.shape[0]
    n = n_s + x_t.shape[0]

    xcat = jnp.concatenate([x_s, x_t], axis=0)
    wl_st = jnp.stack([Wls, Wlt])
    bl_st = jnp.stack([bls[None, :], blt[None, :]])
    at = jnp.stack([a1s, a1d, a2s, a2d], axis=1)

    h1, h2, a = _dense_stage(xcat, wl_st, bl_st, W1, W2, at, n_s)

    src = edge_index[0]
    dst = edge_index[1] + n_s
    loops = jnp.arange(n, dtype=jnp.int32)

    def pack_edges(e_src, e_dst):
        npad = _EP - e_src.shape[0]
        pad_s = jnp.zeros((npad,), jnp.int32)
        pad_d = jnp.full((npad,), _N, jnp.int32)
        es = jnp.concatenate([e_src.astype(jnp.int32), pad_s])
        ed = jnp.concatenate([e_dst.astype(jnp.int32), pad_d])
        return jnp.bitwise_or(es, ed << 16)

    def pad_scores(v):
        return jnp.concatenate([v, jnp.zeros((_NPAD - n,), jnp.float32)])

    sd1 = pack_edges(jnp.concatenate([src, author_edge_index[0], loops]),
                     jnp.concatenate([dst, author_edge_index[1], loops]))
    sd2 = pack_edges(jnp.concatenate([dst, paper_edge_index[0], loops]),
                     jnp.concatenate([src, paper_edge_index[1], loops]))

    zrows = jnp.zeros((_CH // 8, 128), jnp.float32)
    zflat = jnp.zeros((_NPAD // 16,), jnp.float32)
    out1, den1, _, out2, den2, _ = _sc_edge(
        sd1, pad_scores(a[:, 0]), pad_scores(a[:, 1]),
        h1[:, :128], h1[:, 128:],
        sd2, pad_scores(a[:, 2]), pad_scores(a[:, 3]),
        h2[:, :128], h2[:, 128:], zrows, zflat)

    new_x_s = _finish(out2.reshape(_NPAD, _D), den2.reshape(2, _NPAD).T,
                      0, n_s)
    new_x_t = _finish(out1.reshape(_NPAD, _D), den1.reshape(2, _NPAD).T,
                      n_s, n - n_s)
    return (new_x_s, new_x_t)
